# Initial kernel scaffold; baseline (speedup 1.0000x reference)
import jax
import jax.numpy as jnp
from jax.experimental import pallas as pl


def kernel(input_ids, tr):
    raise NotImplementedError("write your pallas kernel here")


# speedup vs baseline: 24.9772x; 24.9772x over previous
import functools

import jax
import jax.numpy as jnp
from jax import lax
from jax.experimental import pallas as pl
from jax.experimental.pallas import tpu as pltpu
from jax.experimental.pallas import tpu_sc as plsc

VOCAB = 1_000_000
WORLD = 4
MY_RANK = 1
PART = VOCAB // WORLD
LO = MY_RANK * PART
HI = LO + PART
EMB = 64
BATCH = 4096
SEQ = 200
NTOK = BATCH * SEQ
NPAD = 1024

NC = 2
NS = 16
NW = NC * NS
PER_W = NTOK // NW
C = 512
NCH = PER_W // C
L = 16


@functools.partial(
    pl.kernel,
    out_type=jax.ShapeDtypeStruct((NTOK, EMB), jnp.float32),
    mesh=plsc.VectorSubcoreMesh(core_axis_name="c", subcore_axis_name="s"),
    compiler_params=pltpu.CompilerParams(use_tc_tiling_on_sc=False),
    scratch_types=[
        pltpu.VMEM((C,), jnp.int32),
        pltpu.VMEM((C,), jnp.int32),
        pltpu.VMEM((C, EMB), jnp.float32),
        pltpu.SemaphoreType.DMA,
    ],
)
def _sc_gather(ids_hbm, tab_hbm, out_hbm, idv, sidv, rows, sem):
    wid = lax.axis_index("s") * NC + lax.axis_index("c")
    base = wid * PER_W

    def chunk(j, carry):
        off = base + j * C
        pltpu.sync_copy(ids_hbm.at[pl.ds(off, C)], idv)

        def vec(i, c2):
            v = idv[pl.ds(i * L, L)]
            m = (v >= LO) & (v < HI)
            sid = jnp.where(m, v - LO, PART + (v & (NPAD - 1)))
            sidv[pl.ds(i * L, L)] = sid
            return c2

        lax.fori_loop(0, C // L, vec, 0)
        pltpu.async_copy(tab_hbm.at[sidv], rows, sem).wait()
        pltpu.sync_copy(rows, out_hbm.at[pl.ds(off, C)])
        return carry

    lax.fori_loop(0, NCH, chunk, 0)


def kernel(input_ids, tr):
    ids = input_ids.reshape(NTOK)
    tab = jnp.concatenate([tr, jnp.zeros((NPAD, EMB), jnp.float32)], axis=0)
    out = _sc_gather(ids, tab)
    return out.reshape(BATCH, SEQ, EMB)
